# Initial kernel scaffold; baseline (speedup 1.0000x reference)
import jax
import jax.numpy as jnp
from jax.experimental import pallas as pl


def kernel(x, edge_index, edge_weight, batch, ddi_edge_index, neg_edge_index, ddi_edge_attr, neg_edge_attr, params):
    raise NotImplementedError("write your pallas kernel here")


# speedup vs baseline: 2.2552x; 2.2552x over previous
import functools
import math

import jax
import jax.numpy as jnp
from jax import lax
from jax.experimental import pallas as pl
from jax.experimental.pallas import tpu as pltpu
from jax.experimental.pallas import tpu_sc as plsc

N = 10000; G = 500; NPG = 20; E = 320000
DF = 128; NH = 64; DH = 64; DE = 16; ED = 2048
K1, K2, K3 = 10, 5, 3

NW = 32
CHR = 4
CH = CHR * 128
E_PAD = 327680
ER = E_PAD // 128

_mesh = functools.partial(
    plsc.VectorSubcoreMesh, core_axis_name="c", subcore_axis_name="s")


def _wid():
    return lax.axis_index("s") * 2 + lax.axis_index("c")


def _zero_rows(buf, nrows, ncol16):
    def z(i, _):
        for c in range(ncol16):
            buf[i, pl.ds(c * 16, 16)] = jnp.zeros((16,), buf.dtype)
        return 0
    lax.fori_loop(0, nrows, z, 0)


def _zero_flat(buf, n):
    def z(i, _):
        buf[pl.ds(i * 16, 16)] = jnp.zeros((16,), buf.dtype)
        return 0
    lax.fori_loop(0, n // 16, z, 0)


def _flush_acc2d(acc_sh, fbuf, out_hbm, cid, sid, rpt):
    off = 0
    while off < rpt:
        blk = min(512, rpt - off)
        lo = sid * rpt + off
        pltpu.sync_copy(acc_sh.at[pl.ds(lo, blk)], fbuf.at[pl.ds(0, blk)])
        pltpu.sync_copy(fbuf.at[pl.ds(0, blk)], out_hbm.at[cid, pl.ds(lo, blk)])
        off += blk


@functools.lru_cache(maxsize=None)
def _make_segsum_vec(n_tbl, n_pad, n_rows, weighted, gathered):
    rpt = n_pad // 16
    nchunks = max(1, n_rows // (NW * CHR))

    def body(tbl_hbm, s2_hbm, d2_hbm, ew_hbm, out_hbm,
             sidx_v, didx_v, ew_v, rows_v, fbuf, acc_sh, gsem):
        cid = lax.axis_index("c"); sid = lax.axis_index("s")
        wid = sid * 2 + cid
        _zero_rows(rows_v, 512, 4)
        _flush_init(acc_sh, rows_v, sid, rpt)
        plsc.subcore_barrier()

        def chunk(i, _):
            rowbase = (i * NW + wid) * CHR

            @pl.when(rowbase < n_rows)
            def _():
                pltpu.sync_copy(d2_hbm.at[pl.ds(rowbase, CHR)], didx_v)
                pltpu.sync_copy(ew_hbm.at[pl.ds(rowbase * 128, CH)], ew_v)
                if gathered:
                    pltpu.sync_copy(s2_hbm.at[pl.ds(rowbase, CHR)], sidx_v)
                    hs = [pltpu.async_copy(
                            tbl_hbm.at[sidx_v.at[j]],
                            rows_v.at[pl.ds(j * 128, 128)], gsem)
                          for j in range(CHR)]
                    for h in hs:
                        h.wait()
                else:
                    pltpu.sync_copy(tbl_hbm.at[pl.ds(rowbase * 128, CH)], rows_v)
                if weighted:
                    def mul(g, _):
                        wv = ew_v[pl.ds(g * 16, 16)]
                        for l in range(16):
                            wb = jnp.full((16,), wv[l], jnp.float32)
                            e = g * 16 + l
                            for c in range(4):
                                rows_v[e, pl.ds(c * 16, 16)] = (
                                    rows_v[e, pl.ds(c * 16, 16)] * wb)
                        return 0
                    lax.fori_loop(0, CH // 16, mul, 0)
                for j in range(CHR):
                    pltpu.sync_copy(rows_v.at[pl.ds(j * 128, 128)],
                                    acc_sh.at[didx_v.at[j]], add=True)
            return 0

        lax.fori_loop(0, nchunks, chunk, 0)
        plsc.subcore_barrier()
        _flush_acc2d(acc_sh, rows_v, out_hbm, cid, sid, rpt)

    def _flush_init(acc_sh, zbuf, sid, rpt):
        off = 0
        while off < rpt:
            blk = min(512, rpt - off)
            pltpu.sync_copy(zbuf.at[pl.ds(0, blk)],
                            acc_sh.at[pl.ds(sid * rpt + off, blk)])
            off += blk

    return pl.kernel(
        body,
        out_type=jax.ShapeDtypeStruct((2, n_pad, 64), jnp.float32),
        mesh=_mesh(),
        compiler_params=pltpu.CompilerParams(
            use_tc_tiling_on_sc=False, needs_layout_passes=False),
        scratch_types=[
            pltpu.VMEM((CHR, 128), jnp.int32),
            pltpu.VMEM((CHR, 128), jnp.int32),
            pltpu.VMEM((CH,), jnp.float32),
            pltpu.VMEM((512, 64), jnp.float32),
            pltpu.VMEM((512, 64), jnp.float32),
            pltpu.VMEM_SHARED((n_pad, 64), jnp.float32),
            pltpu.SemaphoreType.DMA,
        ],
    )


@functools.lru_cache(maxsize=None)
def _make_segsum_scalar(n_pad, n_rows, gathered):
    rpt = n_pad // 16
    nchunks = max(1, n_rows // (NW * CHR))

    def body(r_hbm, s2_hbm, d2_hbm, ew_hbm, out_hbm,
             r_v, sidx_v, didx_v, ew_v, vals_v, fbuf, acc_sh):
        cid = lax.axis_index("c"); sid = lax.axis_index("s")
        wid = sid * 2 + cid
        _zero_flat(fbuf, 512)
        off = 0
        while off < rpt:
            blk = min(512, rpt - off)
            pltpu.sync_copy(fbuf.at[pl.ds(0, blk)],
                            acc_sh.at[pl.ds(sid * rpt + off, blk)])
            off += blk
        if gathered:
            pltpu.sync_copy(r_hbm, r_v)
        plsc.subcore_barrier()

        def chunk(i, _):
            rowbase = (i * NW + wid) * CHR

            @pl.when(rowbase < n_rows)
            def _():
                pltpu.sync_copy(d2_hbm.at[pl.ds(rowbase, CHR)], didx_v)
                pltpu.sync_copy(ew_hbm.at[pl.ds(rowbase * 128, CH)], ew_v)
                if gathered:
                    pltpu.sync_copy(s2_hbm.at[pl.ds(rowbase * 128, CH)], sidx_v)

                    def grp(g, _):
                        sv = sidx_v[pl.ds(g * 16, 16)]
                        wv = ew_v[pl.ds(g * 16, 16)]
                        rv = plsc.load_gather(r_v, [sv])
                        vals_v[pl.ds(g * 16, 16)] = rv * wv
                        return 0
                    lax.fori_loop(0, CH // 16, grp, 0)
                else:
                    def grp(g, _):
                        vals_v[pl.ds(g * 16, 16)] = ew_v[pl.ds(g * 16, 16)]
                        return 0
                    lax.fori_loop(0, CH // 16, grp, 0)
                for j in range(CHR):
                    pltpu.sync_copy(vals_v.at[pl.ds(j * 128, 128)],
                                    acc_sh.at[didx_v.at[j]], add=True)
            return 0

        lax.fori_loop(0, nchunks, chunk, 0)
        plsc.subcore_barrier()
        off = 0
        while off < rpt:
            blk = min(512, rpt - off)
            lo = sid * rpt + off
            pltpu.sync_copy(acc_sh.at[pl.ds(lo, blk)], fbuf.at[pl.ds(0, blk)])
            pltpu.sync_copy(fbuf.at[pl.ds(0, blk)],
                            out_hbm.at[pl.ds(cid * n_pad + lo, blk)])
            off += blk

    return pl.kernel(
        body,
        out_type=jax.ShapeDtypeStruct((2 * n_pad,), jnp.float32),
        mesh=_mesh(),
        compiler_params=pltpu.CompilerParams(needs_layout_passes=False),
        scratch_types=[
            pltpu.VMEM((n_pad,), jnp.float32),
            pltpu.VMEM((CH,), jnp.int32),
            pltpu.VMEM((CHR, 128), jnp.int32),
            pltpu.VMEM((CH,), jnp.float32),
            pltpu.VMEM((CH,), jnp.float32),
            pltpu.VMEM((512,), jnp.float32),
            pltpu.VMEM_SHARED((n_pad,), jnp.float32),
        ],
    )


def _pad_edges(s, d, ew, n):
    pad = E_PAD - E
    spread = (jnp.arange(pad, dtype=jnp.int32) % n)
    s = jnp.concatenate([s.astype(jnp.int32), spread])
    d = jnp.concatenate([d.astype(jnp.int32), spread])
    ew = jnp.concatenate([ew, jnp.zeros((pad,), jnp.float32)])
    return s.reshape(ER, 128), d.reshape(ER, 128), ew


def _seg_vec(tbl, s2, d2, ew, n, n_pad):
    out = _make_segsum_vec(tbl.shape[0], n_pad, s2.shape[0], True, True)(
        tbl, s2, d2, ew)
    return out[0, :n] + out[1, :n]


def _seg_scalar(r_pad, s2flat, d2, ew, n, n_pad, gathered):
    out = _make_segsum_scalar(n_pad, d2.shape[0], gathered)(
        r_pad, s2flat, d2, ew)
    return out[:n] + out[n_pad:n_pad + n]


def _gcn_conv(h, s2, d2, ew, sflat, W, b, n, n_pad):
    xw = h @ W
    deg = _seg_scalar(jnp.zeros((n_pad,), jnp.float32), sflat, d2, ew,
                      n, n_pad, False) + 1.0
    dinv = deg ** -0.5
    y = dinv[:, None] * xw
    agg = _seg_vec(y, s2, d2, ew, n, n_pad)
    return dinv[:, None] * agg + (dinv * dinv)[:, None] * xw + b


def _score(h, s2, d2, ew, sflat, Wrel, Wroot, b, n, n_pad):
    agg = _seg_vec(h, s2, d2, ew, n, n_pad)
    return (agg @ Wrel).reshape(-1) + (h @ Wroot).reshape(-1) + b[0]


def _sag_pool(h, s2, d2, ew, n_per_graph, k, Wrel, Wroot, b, n_pad):
    n_loc = G * n_per_graph
    sflat = s2.reshape(-1)
    score = _score(h, s2, d2, ew, sflat, Wrel, Wroot, b, n_loc, n_pad)
    _, idx = lax.top_k(score.reshape(G, n_per_graph), k)
    kept = (jnp.arange(G)[:, None] * n_per_graph + idx).reshape(-1)
    new_h = h[kept] * jnp.tanh(score[kept])[:, None]
    mapping = jnp.full((n_loc,), -1, jnp.int32).at[kept].set(
        jnp.arange(G * k, dtype=jnp.int32))
    ns = mapping[sflat]; nd = mapping[d2.reshape(-1)]
    keep = (ns >= 0) & (nd >= 0)
    n_new = G * k
    spread = jnp.arange(E_PAD, dtype=jnp.int32) % n_new
    new_s = jnp.where(keep, ns, spread)
    new_d = jnp.where(keep, nd, spread)
    new_ew = jnp.where(keep, ew, 0.0)
    return new_h, new_s.reshape(ER, 128), new_d.reshape(ER, 128), new_ew


def _readout(h, k):
    h3 = h.reshape(G, k, NH)
    return jnp.concatenate([h3.max(axis=1), h3.mean(axis=1)], axis=1)


def _bce_with_logits(l, t):
    return jnp.mean(jnp.maximum(l, 0.0) - l * t + jnp.log1p(jnp.exp(-jnp.abs(l))))


NP1, NP2, NP3 = 10240, 5120, 2560


def kernel(x, edge_index, edge_weight, batch, ddi_edge_index, neg_edge_index,
           ddi_edge_attr, neg_edge_attr, params):
    p = params
    s2, d2, ew = _pad_edges(edge_index[0], edge_index[1], edge_weight, N)

    h = jax.nn.relu(_gcn_conv(x, s2, d2, ew, s2.reshape(-1),
                              p["conv1_W"], p["conv1_b"], N, NP1))
    h, s2, d2, ew = _sag_pool(h, s2, d2, ew, NPG, K1,
                              p["pool1_Wrel"], p["pool1_Wroot"], p["pool1_b"], NP1)
    x1 = _readout(h, K1)
    h = jax.nn.relu(_gcn_conv(h, s2, d2, ew, s2.reshape(-1),
                              p["conv2_W"], p["conv2_b"], G * K1, NP2))
    h, s2, d2, ew = _sag_pool(h, s2, d2, ew, K1, K2,
                              p["pool2_Wrel"], p["pool2_Wroot"], p["pool2_b"], NP2)
    x2 = _readout(h, K2)
    h = jax.nn.relu(_gcn_conv(h, s2, d2, ew, s2.reshape(-1),
                              p["conv3_W"], p["conv3_b"], G * K2, NP3))
    h, s2, d2, ew = _sag_pool(h, s2, d2, ew, K2, K3,
                              p["pool3_Wrel"], p["pool3_Wroot"], p["pool3_b"], NP3)
    x3 = _readout(h, K3)
    out_x = jnp.concatenate([x1, x2, x3], axis=1)

    ds_, dd_ = ddi_edge_index[0], ddi_edge_index[1]
    xs = out_x[ds_]
    Wr = p["nn_W"].reshape(DE, 6 * NH, DH)
    Bmat = p["nn_b"].reshape(6 * NH, DH)
    T = jnp.einsum('ei,aio->eao', xs, Wr)
    msg = jnp.einsum('ea,eao->eo', ddi_edge_attr, T) + xs @ Bmat
    agg = jax.ops.segment_sum(msg, dd_, num_segments=G)
    feat = jax.nn.relu(agg + out_x @ p["conv4_root"] + p["conv4_b"])

    pos_source = feat[ds_]; pos_target = feat[dd_]
    neg_source = feat[neg_edge_index[0]]; neg_target = feat[neg_edge_index[1]]
    pos_feat_x = pos_source @ p["lin1_W"] + p["lin1_b"]
    pos_feat_y = pos_target @ p["lin2_W"] + p["lin2_b"]
    neg_feat_x = neg_source @ p["lin1_W"] + p["lin1_b"]
    neg_feat_y = neg_target @ p["lin2_W"] + p["lin2_b"]
    norm_pos = jnp.sum(pos_feat_x * pos_feat_y, axis=1)
    norm_neg = jnp.sum(neg_feat_x * neg_feat_y, axis=1)
    loss = _bce_with_logits(norm_pos, 1.0) + _bce_with_logits(norm_neg, 0.0)
    return loss, norm_pos, norm_neg, pos_feat_x
